# Initial kernel scaffold; baseline (speedup 1.0000x reference)
import jax
import jax.numpy as jnp
from jax.experimental import pallas as pl


def kernel(x, edge_index, batch, cluster_emb, W, b):
    raise NotImplementedError("write your pallas kernel here")


# speedup vs baseline: 37.0130x; 37.0130x over previous
import functools

import jax
import jax.numpy as jnp
from jax import lax
from jax.experimental import pallas as pl
from jax.experimental.pallas import tpu as pltpu
from jax.experimental.pallas import tpu_sc as plsc

N = 10000
E = 320000
D = 128
K = 16

NC = 2
NS = 16
NW = NC * NS

CHUNK = 128
STEPS = 79
T_TILE = STEPS * CHUNK
E_PAD = NW * T_TILE

N_PAD = 10240
RPT = N_PAD // NS
BLK = 1280
G_TC = N_PAD // BLK

_mesh = plsc.VectorSubcoreMesh(core_axis_name="c", subcore_axis_name="s")


@functools.partial(
    pl.kernel,
    out_type=jax.ShapeDtypeStruct((NC, N_PAD), jnp.float32),
    mesh=_mesh,
    scratch_types=[
        pltpu.VMEM((STEPS, CHUNK), jnp.int32),
        pltpu.VMEM((CHUNK,), jnp.float32),
        pltpu.VMEM((RPT,), jnp.float32),
        pltpu.VMEM_SHARED((N_PAD,), jnp.float32),
    ],
)
def _sc_deg(dst_hbm, out_hbm, idx_v, ones_v, zb_v, deg_sh):
    c = lax.axis_index("c")
    s = lax.axis_index("s")
    wid = c * NS + s
    pltpu.sync_copy(dst_hbm.at[wid], idx_v)
    for i in range(CHUNK // 16):
        ones_v[pl.ds(i * 16, 16)] = jnp.ones((16,), jnp.float32)
    for i in range(RPT // 16):
        zb_v[pl.ds(i * 16, 16)] = jnp.zeros((16,), jnp.float32)
    pltpu.sync_copy(zb_v, deg_sh.at[pl.ds(s * RPT, RPT)])
    plsc.subcore_barrier()

    def body(j, carry):
        pltpu.sync_copy(ones_v, deg_sh.at[idx_v.at[j]], add=True)
        return carry

    lax.fori_loop(0, STEPS, body, 0)
    plsc.subcore_barrier()
    pltpu.sync_copy(deg_sh.at[pl.ds(s * RPT, RPT)],
                    out_hbm.at[c, pl.ds(s * RPT, RPT)])


@functools.partial(
    pl.kernel,
    out_type=jax.ShapeDtypeStruct((NC, N_PAD, K), jnp.float32),
    mesh=_mesh,
    compiler_params=pltpu.CompilerParams(use_tc_tiling_on_sc=False),
    scratch_types=[
        pltpu.VMEM((STEPS, CHUNK), jnp.int32),
        pltpu.VMEM((STEPS, CHUNK), jnp.int32),
        pltpu.VMEM((CHUNK, K), jnp.float32),
        pltpu.VMEM_SHARED((N_PAD, K), jnp.float32),
    ],
)
def _sc_edges(src_hbm, dst_hbm, g_hbm, out_hbm, srcv, dstv, rows_v, agg_sh):
    c = lax.axis_index("c")
    s = lax.axis_index("s")
    wid = c * NS + s
    pltpu.sync_copy(src_hbm.at[wid], srcv)
    pltpu.sync_copy(dst_hbm.at[wid], dstv)
    for i in range(CHUNK):
        rows_v[i] = jnp.zeros((K,), jnp.float32)
    for t in range(RPT // CHUNK):
        pltpu.sync_copy(rows_v, agg_sh.at[pl.ds(s * RPT + t * CHUNK, CHUNK)])
    plsc.subcore_barrier()

    def body(j, carry):
        pltpu.sync_copy(g_hbm.at[srcv.at[j]], rows_v)
        pltpu.sync_copy(rows_v, agg_sh.at[dstv.at[j]], add=True)
        return carry

    lax.fori_loop(0, STEPS, body, 0)
    plsc.subcore_barrier()
    sl = pl.ds(s * RPT, RPT)
    pltpu.sync_copy(agg_sh.at[sl], out_hbm.at[c, sl])


def _tc_prep_body(x_ref, dp_ref, w_ref, ce_ref, g_ref, dinv_ref):
    csum = jnp.sum(ce_ref[...], axis=0, keepdims=True)
    xi = x_ref[...] + csum
    h = jnp.dot(xi, w_ref[...], preferred_element_type=jnp.float32)
    deg = 1.0 + dp_ref[:, 0:1] + dp_ref[:, 1:2]
    dinv = lax.rsqrt(deg)
    g_ref[...] = h * dinv
    dinv_ref[...] = dinv


def _tc_prep(x_pad, dp_t, w, ce):
    return pl.pallas_call(
        _tc_prep_body,
        grid=(G_TC,),
        in_specs=[
            pl.BlockSpec((BLK, D), lambda i: (i, 0)),
            pl.BlockSpec((BLK, NC), lambda i: (i, 0)),
            pl.BlockSpec((D, K), lambda i: (0, 0)),
            pl.BlockSpec((K, D), lambda i: (0, 0)),
        ],
        out_specs=[
            pl.BlockSpec((BLK, K), lambda i: (i, 0)),
            pl.BlockSpec((BLK, 1), lambda i: (i, 0)),
        ],
        out_shape=[
            jax.ShapeDtypeStruct((N_PAD, K), jnp.float32),
            jax.ShapeDtypeStruct((N_PAD, 1), jnp.float32),
        ],
    )(x_pad, dp_t, w, ce)


def _tc_final_body(x_ref, g_ref, dinv_ref, a0_ref, a1_ref, ce_ref, b_ref, o_ref):
    ssum = a0_ref[...] + a1_ref[...]
    logits = dinv_ref[...] * (ssum + g_ref[...]) + b_ref[...]
    m = jnp.max(logits, axis=1, keepdims=True)
    e = jnp.exp(logits - m)
    sm = e / jnp.sum(e, axis=1, keepdims=True)
    p = jnp.dot(sm, ce_ref[...], preferred_element_type=jnp.float32)
    o_ref[...] = x_ref[...] + p


def _tc_final(x_pad, g, dinv, a0, a1, ce, b2):
    return pl.pallas_call(
        _tc_final_body,
        grid=(G_TC,),
        in_specs=[
            pl.BlockSpec((BLK, D), lambda i: (i, 0)),
            pl.BlockSpec((BLK, K), lambda i: (i, 0)),
            pl.BlockSpec((BLK, 1), lambda i: (i, 0)),
            pl.BlockSpec((BLK, K), lambda i: (i, 0)),
            pl.BlockSpec((BLK, K), lambda i: (i, 0)),
            pl.BlockSpec((K, D), lambda i: (0, 0)),
            pl.BlockSpec((1, K), lambda i: (0, 0)),
        ],
        out_specs=pl.BlockSpec((BLK, D), lambda i: (i, 0)),
        out_shape=jax.ShapeDtypeStruct((N_PAD, D), jnp.float32),
    )(x_pad, g, dinv, a0, a1, ce, b2)


@jax.jit
def kernel(x, edge_index, batch, cluster_emb, W, b):
    del batch
    pad = E_PAD - E
    src = jnp.concatenate([edge_index[0], jnp.full((pad,), N, jnp.int32)])
    dst = jnp.concatenate([edge_index[1], jnp.full((pad,), N, jnp.int32)])
    src_r = src.reshape(NW, STEPS, CHUNK)
    dst_r = dst.reshape(NW, STEPS, CHUNK)
    x_pad = jnp.pad(x, ((0, N_PAD - N), (0, 0)))

    deg_parts = _sc_deg(dst_r)
    g, dinv = _tc_prep(x_pad, deg_parts.T, W, cluster_emb)
    agg_parts = _sc_edges(src_r, dst_r, g)
    out_pad = _tc_final(x_pad, g, dinv, agg_parts[0], agg_parts[1],
                        cluster_emb, b.reshape(1, K))
    return out_pad[:N]


# speedup vs baseline: 38.5412x; 1.0413x over previous
import functools

import jax
import jax.numpy as jnp
from jax import lax
from jax.experimental import pallas as pl
from jax.experimental.pallas import tpu as pltpu
from jax.experimental.pallas import tpu_sc as plsc

N = 10000
E = 320000
D = 128
K = 16

NC = 2
NS = 16
NW = NC * NS

CH = 80
STEPS = 125
T_TILE = STEPS * CH

N_PAD = 10240
RPT = N_PAD // NS
BLK = 1280
G_TC = N_PAD // BLK

_mesh = plsc.VectorSubcoreMesh(core_axis_name="c", subcore_axis_name="s")
_sc_params = pltpu.CompilerParams(use_tc_tiling_on_sc=False)


@functools.partial(
    pl.kernel,
    out_type=jax.ShapeDtypeStruct((NC, N_PAD), jnp.float32),
    mesh=_mesh,
    compiler_params=_sc_params,
    scratch_types=[
        pltpu.VMEM((STEPS, CH), jnp.int32),
        pltpu.VMEM((CH,), jnp.float32),
        pltpu.VMEM_SHARED((N_PAD,), jnp.float32),
    ],
)
def _sc_deg(e3_hbm, ones_hbm, z1_hbm, out_hbm, idx_v, ones_v, deg_sh):
    c = lax.axis_index("c")
    s = lax.axis_index("s")
    wid = c * NS + s
    pltpu.sync_copy(e3_hbm.at[1, pl.ds(wid * STEPS, STEPS)], idx_v)
    pltpu.sync_copy(ones_hbm, ones_v)
    pltpu.sync_copy(z1_hbm, deg_sh.at[pl.ds(s * RPT, RPT)])
    plsc.subcore_barrier()

    def body(j, carry):
        pltpu.sync_copy(ones_v, deg_sh.at[idx_v.at[j]], add=True)
        return carry

    lax.fori_loop(0, STEPS, body, 0)
    plsc.subcore_barrier()
    pltpu.sync_copy(deg_sh.at[pl.ds(s * RPT, RPT)],
                    out_hbm.at[c, pl.ds(s * RPT, RPT)])


@functools.partial(
    pl.kernel,
    out_type=jax.ShapeDtypeStruct((NC, N_PAD, K), jnp.float32),
    mesh=_mesh,
    compiler_params=_sc_params,
    scratch_types=[
        pltpu.VMEM((STEPS, CH), jnp.int32),
        pltpu.VMEM((STEPS, CH), jnp.int32),
        pltpu.VMEM((CH, K), jnp.float32),
        pltpu.VMEM_SHARED((N_PAD, K), jnp.float32),
    ],
)
def _sc_edges(e3_hbm, g_hbm, z16_hbm, out_hbm, srcv, dstv, rows_v, agg_sh):
    c = lax.axis_index("c")
    s = lax.axis_index("s")
    wid = c * NS + s
    pltpu.sync_copy(e3_hbm.at[0, pl.ds(wid * STEPS, STEPS)], srcv)
    pltpu.sync_copy(e3_hbm.at[1, pl.ds(wid * STEPS, STEPS)], dstv)
    pltpu.sync_copy(z16_hbm, agg_sh.at[pl.ds(s * RPT, RPT)])
    plsc.subcore_barrier()

    def body(j, carry):
        pltpu.sync_copy(g_hbm.at[srcv.at[j]], rows_v)
        pltpu.sync_copy(rows_v, agg_sh.at[dstv.at[j]], add=True)
        return carry

    lax.fori_loop(0, STEPS, body, 0)
    plsc.subcore_barrier()
    sl = pl.ds(s * RPT, RPT)
    pltpu.sync_copy(agg_sh.at[sl], out_hbm.at[c, sl])


def _tc_prep_body(x_ref, dp_ref, w_ref, ce_ref, g_ref, dinv_ref):
    csum = jnp.sum(ce_ref[...], axis=0, keepdims=True)
    xi = x_ref[...] + csum
    h = jnp.dot(xi, w_ref[...], preferred_element_type=jnp.float32)
    deg = 1.0 + lax.dot_general(
        dp_ref[...], jnp.ones((NC, 1), jnp.float32),
        (((0,), (0,)), ((), ())), preferred_element_type=jnp.float32)
    dinv = lax.rsqrt(deg)
    g_ref[...] = h * dinv
    dinv_ref[...] = dinv


def _tc_prep(x, dp, w, ce):
    return pl.pallas_call(
        _tc_prep_body,
        grid=(G_TC,),
        in_specs=[
            pl.BlockSpec((BLK, D), lambda i: (i, 0)),
            pl.BlockSpec((NC, BLK), lambda i: (0, i)),
            pl.BlockSpec((D, K), lambda i: (0, 0)),
            pl.BlockSpec((K, D), lambda i: (0, 0)),
        ],
        out_specs=[
            pl.BlockSpec((BLK, K), lambda i: (i, 0)),
            pl.BlockSpec((BLK, 1), lambda i: (i, 0)),
        ],
        out_shape=[
            jax.ShapeDtypeStruct((N_PAD, K), jnp.float32),
            jax.ShapeDtypeStruct((N_PAD, 1), jnp.float32),
        ],
    )(x, dp, w, ce)


def _tc_final_body(x_ref, g_ref, dinv_ref, agg_ref, ce_ref, b_ref, o_ref):
    ssum = agg_ref[0] + agg_ref[1]
    logits = dinv_ref[...] * (ssum + g_ref[...]) + b_ref[...]
    m = jnp.max(logits, axis=1, keepdims=True)
    e = jnp.exp(logits - m)
    sm = e / jnp.sum(e, axis=1, keepdims=True)
    p = jnp.dot(sm, ce_ref[...], preferred_element_type=jnp.float32)
    o_ref[...] = x_ref[...] + p


def _tc_final(x, g, dinv, agg, ce, b2):
    return pl.pallas_call(
        _tc_final_body,
        grid=(G_TC,),
        in_specs=[
            pl.BlockSpec((BLK, D), lambda i: (i, 0)),
            pl.BlockSpec((BLK, K), lambda i: (i, 0)),
            pl.BlockSpec((BLK, 1), lambda i: (i, 0)),
            pl.BlockSpec((NC, BLK, K), lambda i: (0, i, 0)),
            pl.BlockSpec((K, D), lambda i: (0, 0)),
            pl.BlockSpec((1, K), lambda i: (0, 0)),
        ],
        out_specs=pl.BlockSpec((BLK, D), lambda i: (i, 0)),
        out_shape=jax.ShapeDtypeStruct((N, D), jnp.float32),
    )(x, g, dinv, agg, ce, b2)


@jax.jit
def kernel(x, edge_index, batch, cluster_emb, W, b):
    del batch
    e3 = edge_index.reshape(2, NW * STEPS, CH)
    ones1 = jnp.ones((CH,), jnp.float32)
    z1 = jnp.zeros((RPT,), jnp.float32)
    z16 = jnp.zeros((RPT, K), jnp.float32)

    deg2 = _sc_deg(e3, ones1, z1)
    g, dinv = _tc_prep(x, deg2, W, cluster_emb)
    agg = _sc_edges(e3, g, z16)
    return _tc_final(x, g, dinv, agg, cluster_emb, b.reshape(1, K))


# speedup vs baseline: 51.5506x; 1.3375x over previous
import functools

import jax
import jax.numpy as jnp
from jax import lax
from jax.experimental import pallas as pl
from jax.experimental.pallas import tpu as pltpu
from jax.experimental.pallas import tpu_sc as plsc

N = 10000
E = 320000
D = 128
K = 16

NC = 2
NS = 16
NW = NC * NS

CH = 128
NCHUNK = E // CH
BASE_STEPS = NCHUNK // NW
EXTRA = NCHUNK - BASE_STEPS * NW
MAX_STEPS = BASE_STEPS + 1

N_PAD = 10240
RPT = N_PAD // NS
BLK = 1280
G_TC = N_PAD // BLK

_mesh = plsc.VectorSubcoreMesh(core_axis_name="c", subcore_axis_name="s")
_sc_params = pltpu.CompilerParams(use_tc_tiling_on_sc=False)


def _chunk_start(wid):
    return BASE_STEPS * wid + jnp.minimum(wid, EXTRA)


@functools.partial(
    pl.kernel,
    out_type=jax.ShapeDtypeStruct((NC, N_PAD), jnp.float32),
    mesh=_mesh,
    compiler_params=_sc_params,
    scratch_types=[
        pltpu.VMEM((MAX_STEPS, CH), jnp.int32),
        pltpu.VMEM((CH,), jnp.float32),
        pltpu.VMEM((RPT,), jnp.float32),
        pltpu.VMEM_SHARED((N_PAD,), jnp.float32),
    ],
)
def _sc_deg(e3_hbm, out_hbm, idx_v, ones_v, zb_v, deg_sh):
    c = lax.axis_index("c")
    s = lax.axis_index("s")
    wid = c * NS + s
    start = _chunk_start(wid)
    nsteps = BASE_STEPS + jnp.where(wid < EXTRA, 1, 0)
    pltpu.sync_copy(e3_hbm.at[1, pl.ds(start, BASE_STEPS)],
                    idx_v.at[pl.ds(0, BASE_STEPS)])

    @pl.when(wid < EXTRA)
    def _():
        pltpu.sync_copy(e3_hbm.at[1, pl.ds(start + BASE_STEPS, 1)],
                        idx_v.at[pl.ds(BASE_STEPS, 1)])

    for i in range(CH // 16):
        ones_v[pl.ds(i * 16, 16)] = jnp.ones((16,), jnp.float32)
    for i in range(RPT // 16):
        zb_v[pl.ds(i * 16, 16)] = jnp.zeros((16,), jnp.float32)
    pltpu.sync_copy(zb_v, deg_sh.at[pl.ds(s * RPT, RPT)])
    plsc.subcore_barrier()

    def body(j, carry):
        pltpu.sync_copy(ones_v, deg_sh.at[idx_v.at[j]], add=True)
        return carry

    lax.fori_loop(0, nsteps, body, 0)
    plsc.subcore_barrier()
    pltpu.sync_copy(deg_sh.at[pl.ds(s * RPT, RPT)],
                    out_hbm.at[c, pl.ds(s * RPT, RPT)])


@functools.partial(
    pl.kernel,
    out_type=jax.ShapeDtypeStruct((NC, N_PAD, 128), jnp.float32),
    mesh=_mesh,
    compiler_params=_sc_params,
    scratch_types=[
        pltpu.VMEM((MAX_STEPS, CH), jnp.int32),
        pltpu.VMEM((MAX_STEPS, CH), jnp.int32),
        pltpu.VMEM((CH, K), jnp.float32),
        pltpu.VMEM_SHARED((N_PAD, K), jnp.float32),
    ],
)
def _sc_edges(e3_hbm, g8_hbm, out_hbm, srcv, dstv, rows_v, agg_sh):
    c = lax.axis_index("c")
    s = lax.axis_index("s")
    wid = c * NS + s
    start = _chunk_start(wid)
    nsteps = BASE_STEPS + jnp.where(wid < EXTRA, 1, 0)
    pltpu.sync_copy(e3_hbm.at[0, pl.ds(start, BASE_STEPS)],
                    srcv.at[pl.ds(0, BASE_STEPS)])
    pltpu.sync_copy(e3_hbm.at[1, pl.ds(start, BASE_STEPS)],
                    dstv.at[pl.ds(0, BASE_STEPS)])

    @pl.when(wid < EXTRA)
    def _():
        pltpu.sync_copy(e3_hbm.at[0, pl.ds(start + BASE_STEPS, 1)],
                        srcv.at[pl.ds(BASE_STEPS, 1)])
        pltpu.sync_copy(e3_hbm.at[1, pl.ds(start + BASE_STEPS, 1)],
                        dstv.at[pl.ds(BASE_STEPS, 1)])

    def scale(j, carry):
        for i in range(CH // 16):
            sl = pl.ds(i * 16, 16)
            srcv[j, sl] = lax.shift_left(srcv[j, sl], 3)
        return carry

    lax.fori_loop(0, MAX_STEPS, scale, 0)

    for i in range(CH):
        rows_v[i] = jnp.zeros((K,), jnp.float32)
    for t in range(RPT // CH):
        pltpu.sync_copy(rows_v, agg_sh.at[pl.ds(s * RPT + t * CH, CH)])
    plsc.subcore_barrier()

    def body(j, carry):
        pltpu.sync_copy(g8_hbm.at[srcv.at[j]], rows_v)
        pltpu.sync_copy(rows_v, agg_sh.at[dstv.at[j]], add=True)
        return carry

    lax.fori_loop(0, nsteps, body, 0)
    plsc.subcore_barrier()
    sl = pl.ds(s * RPT, RPT)
    pltpu.sync_copy(agg_sh.at[sl], out_hbm.at[c, sl, pl.ds(0, K)])


def _tc_prep_body(x_ref, dp_ref, w_ref, ce_ref, g_ref, dinv_ref):
    csum = jnp.sum(ce_ref[...], axis=0, keepdims=True)
    xi = x_ref[...] + csum
    h = jnp.dot(xi, w_ref[...], preferred_element_type=jnp.float32)
    deg = 1.0 + lax.dot_general(
        dp_ref[...], jnp.ones((NC, 1), jnp.float32),
        (((0,), (0,)), ((), ())), preferred_element_type=jnp.float32)
    dinv = lax.rsqrt(deg)
    g_ref[...] = jnp.concatenate(
        [h * dinv, jnp.zeros((BLK, 128 - K), jnp.float32)], axis=1)
    dinv_ref[...] = dinv


def _tc_prep(x, dp, w, ce):
    return pl.pallas_call(
        _tc_prep_body,
        grid=(G_TC,),
        in_specs=[
            pl.BlockSpec((BLK, D), lambda i: (i, 0)),
            pl.BlockSpec((NC, BLK), lambda i: (0, i)),
            pl.BlockSpec((D, K), lambda i: (0, 0)),
            pl.BlockSpec((K, D), lambda i: (0, 0)),
        ],
        out_specs=[
            pl.BlockSpec((BLK, 128), lambda i: (i, 0)),
            pl.BlockSpec((BLK, 1), lambda i: (i, 0)),
        ],
        out_shape=[
            jax.ShapeDtypeStruct((N_PAD, 128), jnp.float32),
            jax.ShapeDtypeStruct((N_PAD, 1), jnp.float32),
        ],
    )(x, dp, w, ce)


def _tc_final_body(x_ref, g_ref, dinv_ref, agg_ref, ce_ref, b_ref, o_ref):
    a = agg_ref[...]
    ssum = a[0, :, 0:K] + a[1, :, 0:K]
    g = g_ref[:, 0:K]
    logits = dinv_ref[...] * (ssum + g) + b_ref[...]
    m = jnp.max(logits, axis=1, keepdims=True)
    e = jnp.exp(logits - m)
    sm = e / jnp.sum(e, axis=1, keepdims=True)
    p = jnp.dot(sm, ce_ref[...], preferred_element_type=jnp.float32)
    o_ref[...] = x_ref[...] + p


def _tc_final(x, g, dinv, agg, ce, b2):
    return pl.pallas_call(
        _tc_final_body,
        grid=(G_TC,),
        in_specs=[
            pl.BlockSpec((BLK, D), lambda i: (i, 0)),
            pl.BlockSpec((BLK, 128), lambda i: (i, 0)),
            pl.BlockSpec((BLK, 1), lambda i: (i, 0)),
            pl.BlockSpec((NC, BLK, 128), lambda i: (0, i, 0)),
            pl.BlockSpec((K, D), lambda i: (0, 0)),
            pl.BlockSpec((1, K), lambda i: (0, 0)),
        ],
        out_specs=pl.BlockSpec((BLK, D), lambda i: (i, 0)),
        out_shape=jax.ShapeDtypeStruct((N, D), jnp.float32),
    )(x, g, dinv, agg, ce, b2)


@jax.jit
def kernel(x, edge_index, batch, cluster_emb, W, b):
    del batch
    e3 = edge_index.reshape(2, NCHUNK, CH)

    deg2 = _sc_deg(e3)
    g, dinv = _tc_prep(x, deg2, W, cluster_emb)
    g8 = g.reshape(N_PAD * 8, K)
    agg = _sc_edges(e3, g8)
    return _tc_final(x, g, dinv, agg, cluster_emb, b.reshape(1, K))


# speedup vs baseline: 53.2920x; 1.0338x over previous
import functools

import jax
import jax.numpy as jnp
from jax import lax
from jax.experimental import pallas as pl
from jax.experimental.pallas import tpu as pltpu
from jax.experimental.pallas import tpu_sc as plsc

N = 10000
E = 320000
D = 128
K = 16

NC = 2
NS = 16
NW = NC * NS

CH = 128
NCHUNK = E // CH
BASE_STEPS = NCHUNK // NW
EXTRA = NCHUNK - BASE_STEPS * NW
MAX_STEPS = BASE_STEPS + 1

N_PAD = 10240
RPT = N_PAD // NS
BLK = 1280
G_TC = N_PAD // BLK

_mesh = plsc.VectorSubcoreMesh(core_axis_name="c", subcore_axis_name="s")
_sc_params = pltpu.CompilerParams(use_tc_tiling_on_sc=False)


def _chunk_start(wid):
    return BASE_STEPS * wid + jnp.minimum(wid, EXTRA)


@functools.partial(
    pl.kernel,
    out_type=jax.ShapeDtypeStruct((NC, N_PAD), jnp.float32),
    mesh=_mesh,
    compiler_params=_sc_params,
    scratch_types=[
        pltpu.VMEM((MAX_STEPS, CH), jnp.int32),
        pltpu.VMEM((CH,), jnp.float32),
        pltpu.VMEM((RPT,), jnp.float32),
        pltpu.VMEM_SHARED((N_PAD,), jnp.float32),
        pltpu.SemaphoreType.DMA,
    ],
)
def _sc_deg(e3_hbm, out_hbm, idx_v, ones_v, zb_v, deg_sh, sem):
    c = lax.axis_index("c")
    s = lax.axis_index("s")
    wid = c * NS + s
    start = _chunk_start(wid)
    nsteps = BASE_STEPS + jnp.where(wid < EXTRA, 1, 0)
    pltpu.sync_copy(e3_hbm.at[1, pl.ds(start, BASE_STEPS)],
                    idx_v.at[pl.ds(0, BASE_STEPS)])

    @pl.when(wid < EXTRA)
    def _():
        pltpu.sync_copy(e3_hbm.at[1, pl.ds(start + BASE_STEPS, 1)],
                        idx_v.at[pl.ds(BASE_STEPS, 1)])

    for i in range(CH // 16):
        ones_v[pl.ds(i * 16, 16)] = jnp.ones((16,), jnp.float32)
    for i in range(RPT // 16):
        zb_v[pl.ds(i * 16, 16)] = jnp.zeros((16,), jnp.float32)
    pltpu.sync_copy(zb_v, deg_sh.at[pl.ds(s * RPT, RPT)])
    plsc.subcore_barrier()

    GK = 13
    GROUPS = BASE_STEPS // GK
    for gidx in range(GROUPS):
        for b in range(GK):
            pltpu.async_copy(ones_v, deg_sh.at[idx_v.at[gidx * GK + b]],
                             sem, add=True)
        if gidx >= 1:
            for b in range(GK):
                pltpu.make_async_copy(
                    ones_v, deg_sh.at[idx_v.at[b]], sem).wait()
    for b in range(GK):
        pltpu.make_async_copy(ones_v, deg_sh.at[idx_v.at[b]], sem).wait()

    @pl.when(wid < EXTRA)
    def _():
        pltpu.sync_copy(ones_v, deg_sh.at[idx_v.at[BASE_STEPS]], add=True)

    plsc.subcore_barrier()
    pltpu.sync_copy(deg_sh.at[pl.ds(s * RPT, RPT)],
                    out_hbm.at[c, pl.ds(s * RPT, RPT)])


@functools.partial(
    pl.kernel,
    out_type=jax.ShapeDtypeStruct((NC, N_PAD, 128), jnp.float32),
    mesh=_mesh,
    compiler_params=_sc_params,
    scratch_types=[
        pltpu.VMEM((MAX_STEPS, CH), jnp.int32),
        pltpu.VMEM((MAX_STEPS, CH), jnp.int32),
        pltpu.VMEM((CH, K), jnp.float32),
        pltpu.VMEM_SHARED((N_PAD, K), jnp.float32),
    ],
)
def _sc_edges(e3_hbm, g8_hbm, out_hbm, srcv, dstv, rows_v, agg_sh):
    c = lax.axis_index("c")
    s = lax.axis_index("s")
    wid = c * NS + s
    start = _chunk_start(wid)
    nsteps = BASE_STEPS + jnp.where(wid < EXTRA, 1, 0)
    pltpu.sync_copy(e3_hbm.at[0, pl.ds(start, BASE_STEPS)],
                    srcv.at[pl.ds(0, BASE_STEPS)])
    pltpu.sync_copy(e3_hbm.at[1, pl.ds(start, BASE_STEPS)],
                    dstv.at[pl.ds(0, BASE_STEPS)])

    @pl.when(wid < EXTRA)
    def _():
        pltpu.sync_copy(e3_hbm.at[0, pl.ds(start + BASE_STEPS, 1)],
                        srcv.at[pl.ds(BASE_STEPS, 1)])
        pltpu.sync_copy(e3_hbm.at[1, pl.ds(start + BASE_STEPS, 1)],
                        dstv.at[pl.ds(BASE_STEPS, 1)])

    def scale(j, carry):
        for i in range(CH // 16):
            sl = pl.ds(i * 16, 16)
            srcv[j, sl] = lax.shift_left(srcv[j, sl], 3)
        return carry

    lax.fori_loop(0, MAX_STEPS, scale, 0)

    for i in range(CH):
        rows_v[i] = jnp.zeros((K,), jnp.float32)
    for t in range(RPT // CH):
        pltpu.sync_copy(rows_v, agg_sh.at[pl.ds(s * RPT + t * CH, CH)])
    plsc.subcore_barrier()

    def body(j, carry):
        pltpu.sync_copy(g8_hbm.at[srcv.at[j]], rows_v)
        pltpu.sync_copy(rows_v, agg_sh.at[dstv.at[j]], add=True)
        return carry

    lax.fori_loop(0, nsteps, body, 0)
    plsc.subcore_barrier()
    sl = pl.ds(s * RPT, RPT)
    pltpu.sync_copy(agg_sh.at[sl], out_hbm.at[c, sl, pl.ds(0, K)])


def _tc_prep_body(x_ref, dp_ref, w_ref, ce_ref, g_ref, dinv_ref):
    csum = jnp.sum(ce_ref[...], axis=0, keepdims=True)
    xi = x_ref[...] + csum
    h = jnp.dot(xi, w_ref[...], preferred_element_type=jnp.float32)
    deg = 1.0 + lax.dot_general(
        dp_ref[...], jnp.ones((NC, 1), jnp.float32),
        (((0,), (0,)), ((), ())), preferred_element_type=jnp.float32)
    dinv = lax.rsqrt(deg)
    g_ref[...] = jnp.concatenate(
        [h * dinv, jnp.zeros((BLK, 128 - K), jnp.float32)], axis=1)
    dinv_ref[...] = dinv


def _tc_prep(x, dp, w, ce):
    return pl.pallas_call(
        _tc_prep_body,
        grid=(G_TC,),
        in_specs=[
            pl.BlockSpec((BLK, D), lambda i: (i, 0)),
            pl.BlockSpec((NC, BLK), lambda i: (0, i)),
            pl.BlockSpec((D, K), lambda i: (0, 0)),
            pl.BlockSpec((K, D), lambda i: (0, 0)),
        ],
        out_specs=[
            pl.BlockSpec((BLK, 128), lambda i: (i, 0)),
            pl.BlockSpec((BLK, 1), lambda i: (i, 0)),
        ],
        out_shape=[
            jax.ShapeDtypeStruct((N_PAD, 128), jnp.float32),
            jax.ShapeDtypeStruct((N_PAD, 1), jnp.float32),
        ],
    )(x, dp, w, ce)


def _tc_final_body(x_ref, g_ref, dinv_ref, agg_ref, ce_ref, b_ref, o_ref):
    a = agg_ref[...]
    ssum = a[0, :, 0:K] + a[1, :, 0:K]
    logits = dinv_ref[...] * (ssum + g_ref[:, 0:K]) + b_ref[...]
    m = jnp.max(logits, axis=1, keepdims=True)
    e = jnp.exp(logits - m)
    sm = e / jnp.sum(e, axis=1, keepdims=True)
    p = jnp.dot(sm, ce_ref[...], preferred_element_type=jnp.float32)
    o_ref[...] = x_ref[...] + p


def _tc_final(x, g, dinv, agg, ce, b2):
    return pl.pallas_call(
        _tc_final_body,
        grid=(G_TC,),
        in_specs=[
            pl.BlockSpec((BLK, D), lambda i: (i, 0)),
            pl.BlockSpec((BLK, 128), lambda i: (i, 0)),
            pl.BlockSpec((BLK, 1), lambda i: (i, 0)),
            pl.BlockSpec((NC, BLK, 128), lambda i: (0, i, 0)),
            pl.BlockSpec((K, D), lambda i: (0, 0)),
            pl.BlockSpec((1, K), lambda i: (0, 0)),
        ],
        out_specs=pl.BlockSpec((BLK, D), lambda i: (i, 0)),
        out_shape=jax.ShapeDtypeStruct((N, D), jnp.float32),
    )(x, g, dinv, agg, ce, b2)


@jax.jit
def kernel(x, edge_index, batch, cluster_emb, W, b):
    del batch
    e3 = edge_index.reshape(2, NCHUNK, CH)

    deg2 = _sc_deg(e3)
    g, dinv = _tc_prep(x, deg2, W, cluster_emb)
    g8 = g.reshape(N_PAD * 8, K)
    agg = _sc_edges(e3, g8)
    return _tc_final(x, g, dinv, agg, cluster_emb, b.reshape(1, K))


# speedup vs baseline: 87.1608x; 1.6355x over previous
import functools

import jax
import jax.numpy as jnp
from jax import lax
from jax.experimental import pallas as pl
from jax.experimental.pallas import tpu as pltpu
from jax.experimental.pallas import tpu_sc as plsc

N = 10000
E = 320000
D = 128
K = 16

NC = 2
NS = 16
NW = NC * NS

CH = 128
NCHUNK = E // CH
BASE_STEPS = NCHUNK // NW
EXTRA = NCHUNK - BASE_STEPS * NW
MAX_STEPS = BASE_STEPS + 1

N_PAD = 10240
RPT = N_PAD // NS
BLK = 1280
G_TC = N_PAD // BLK

_mesh = plsc.VectorSubcoreMesh(core_axis_name="c", subcore_axis_name="s")
_sc_params = pltpu.CompilerParams(use_tc_tiling_on_sc=False)


def _chunk_start(wid):
    return BASE_STEPS * wid + jnp.minimum(wid, EXTRA)


@functools.partial(
    pl.kernel,
    out_type=jax.ShapeDtypeStruct((NC, N_PAD), jnp.float32),
    mesh=_mesh,
    compiler_params=_sc_params,
    scratch_types=[
        pltpu.VMEM((MAX_STEPS, CH), jnp.int32),
        pltpu.VMEM((CH,), jnp.float32),
        pltpu.VMEM((RPT,), jnp.float32),
        pltpu.VMEM_SHARED((N_PAD,), jnp.float32),
        pltpu.SemaphoreType.DMA,
    ],
)
def _sc_deg(e3_hbm, out_hbm, idx_v, ones_v, zb_v, deg_sh, sem):
    c = lax.axis_index("c")
    s = lax.axis_index("s")
    wid = c * NS + s
    start = _chunk_start(wid)
    nsteps = BASE_STEPS + jnp.where(wid < EXTRA, 1, 0)
    pltpu.sync_copy(e3_hbm.at[1, pl.ds(start, BASE_STEPS)],
                    idx_v.at[pl.ds(0, BASE_STEPS)])

    @pl.when(wid < EXTRA)
    def _():
        pltpu.sync_copy(e3_hbm.at[1, pl.ds(start + BASE_STEPS, 1)],
                        idx_v.at[pl.ds(BASE_STEPS, 1)])

    for i in range(CH // 16):
        ones_v[pl.ds(i * 16, 16)] = jnp.ones((16,), jnp.float32)
    for i in range(RPT // 16):
        zb_v[pl.ds(i * 16, 16)] = jnp.zeros((16,), jnp.float32)
    pltpu.sync_copy(zb_v, deg_sh.at[pl.ds(s * RPT, RPT)])
    plsc.subcore_barrier()

    GK = 13
    GROUPS = BASE_STEPS // GK
    for gidx in range(GROUPS):
        for b in range(GK):
            pltpu.async_copy(ones_v, deg_sh.at[idx_v.at[gidx * GK + b]],
                             sem, add=True)
        if gidx >= 1:
            for b in range(GK):
                pltpu.make_async_copy(
                    ones_v, deg_sh.at[idx_v.at[b]], sem).wait()
    for b in range(GK):
        pltpu.make_async_copy(ones_v, deg_sh.at[idx_v.at[b]], sem).wait()

    @pl.when(wid < EXTRA)
    def _():
        pltpu.sync_copy(ones_v, deg_sh.at[idx_v.at[BASE_STEPS]], add=True)

    plsc.subcore_barrier()
    pltpu.sync_copy(deg_sh.at[pl.ds(s * RPT, RPT)],
                    out_hbm.at[c, pl.ds(s * RPT, RPT)])


@functools.partial(
    pl.kernel,
    out_type=jax.ShapeDtypeStruct((NC, N_PAD, 128), jnp.float32),
    mesh=_mesh,
    compiler_params=_sc_params,
    scratch_types=[
        pltpu.VMEM((MAX_STEPS, CH), jnp.int32),
        pltpu.VMEM((MAX_STEPS, CH), jnp.int32),
        pltpu.VMEM((2, 13, CH, K), jnp.float32),
        pltpu.VMEM_SHARED((N_PAD, K), jnp.float32),
        pltpu.SemaphoreType.DMA,
        pltpu.SemaphoreType.DMA,
        pltpu.SemaphoreType.DMA,
    ],
)
def _sc_edges(e3_hbm, g8_hbm, out_hbm, srcv, dstv, rows_v, agg_sh,
              gsem, ssem0, ssem1):
    c = lax.axis_index("c")
    s = lax.axis_index("s")
    wid = c * NS + s
    start = _chunk_start(wid)
    nsteps = BASE_STEPS + jnp.where(wid < EXTRA, 1, 0)
    pltpu.sync_copy(e3_hbm.at[0, pl.ds(start, BASE_STEPS)],
                    srcv.at[pl.ds(0, BASE_STEPS)])
    pltpu.sync_copy(e3_hbm.at[1, pl.ds(start, BASE_STEPS)],
                    dstv.at[pl.ds(0, BASE_STEPS)])

    @pl.when(wid < EXTRA)
    def _():
        pltpu.sync_copy(e3_hbm.at[0, pl.ds(start + BASE_STEPS, 1)],
                        srcv.at[pl.ds(BASE_STEPS, 1)])
        pltpu.sync_copy(e3_hbm.at[1, pl.ds(start + BASE_STEPS, 1)],
                        dstv.at[pl.ds(BASE_STEPS, 1)])

    def scale(j, carry):
        for i in range(CH // 16):
            sl = pl.ds(i * 16, 16)
            srcv[j, sl] = lax.shift_left(srcv[j, sl], 3)
        return carry

    lax.fori_loop(0, MAX_STEPS, scale, 0)

    for i in range(CH):
        rows_v[0, 0, i] = jnp.zeros((K,), jnp.float32)
    for t in range(RPT // CH):
        pltpu.sync_copy(rows_v.at[0, 0],
                        agg_sh.at[pl.ds(s * RPT + t * CH, CH)])
    plsc.subcore_barrier()

    GK = 13
    ssems = (ssem0, ssem1)

    def super_body(t, carry):
        for p in range(2):
            gbase = (2 * t + p) * GK

            @pl.when(t > 0)
            def _():
                for b in range(GK):
                    pltpu.make_async_copy(
                        rows_v.at[p, b], agg_sh.at[dstv.at[b]],
                        ssems[p]).wait()

            for b in range(GK):
                pltpu.async_copy(g8_hbm.at[srcv.at[gbase + b]],
                                 rows_v.at[p, b], gsem)
            for b in range(GK):
                pltpu.make_async_copy(g8_hbm.at[srcv.at[gbase + b]],
                                      rows_v.at[p, b], gsem).wait()
            for b in range(GK):
                pltpu.async_copy(rows_v.at[p, b],
                                 agg_sh.at[dstv.at[gbase + b]],
                                 ssems[p], add=True)
        return carry

    lax.fori_loop(0, BASE_STEPS // (2 * GK), super_body, 0)
    for p in range(2):
        for b in range(GK):
            pltpu.make_async_copy(rows_v.at[p, b], agg_sh.at[dstv.at[b]],
                                  ssems[p]).wait()

    @pl.when(wid < EXTRA)
    def _():
        pltpu.sync_copy(g8_hbm.at[srcv.at[BASE_STEPS]], rows_v.at[0, 0])
        pltpu.sync_copy(rows_v.at[0, 0], agg_sh.at[dstv.at[BASE_STEPS]],
                        add=True)

    plsc.subcore_barrier()
    sl = pl.ds(s * RPT, RPT)
    pltpu.sync_copy(agg_sh.at[sl], out_hbm.at[c, sl, pl.ds(0, K)])


def _tc_prep_body(x_ref, dp_ref, w_ref, ce_ref, g_ref, dinv_ref):
    csum = jnp.sum(ce_ref[...], axis=0, keepdims=True)
    xi = x_ref[...] + csum
    h = jnp.dot(xi, w_ref[...], preferred_element_type=jnp.float32)
    deg = 1.0 + lax.dot_general(
        dp_ref[...], jnp.ones((NC, 1), jnp.float32),
        (((0,), (0,)), ((), ())), preferred_element_type=jnp.float32)
    dinv = lax.rsqrt(deg)
    g_ref[...] = jnp.concatenate(
        [h * dinv, jnp.zeros((BLK, 128 - K), jnp.float32)], axis=1)
    dinv_ref[...] = dinv


def _tc_prep(x, dp, w, ce):
    return pl.pallas_call(
        _tc_prep_body,
        grid=(G_TC,),
        in_specs=[
            pl.BlockSpec((BLK, D), lambda i: (i, 0)),
            pl.BlockSpec((NC, BLK), lambda i: (0, i)),
            pl.BlockSpec((D, K), lambda i: (0, 0)),
            pl.BlockSpec((K, D), lambda i: (0, 0)),
        ],
        out_specs=[
            pl.BlockSpec((BLK, 128), lambda i: (i, 0)),
            pl.BlockSpec((BLK, 1), lambda i: (i, 0)),
        ],
        out_shape=[
            jax.ShapeDtypeStruct((N_PAD, 128), jnp.float32),
            jax.ShapeDtypeStruct((N_PAD, 1), jnp.float32),
        ],
    )(x, dp, w, ce)


def _tc_final_body(x_ref, g_ref, dinv_ref, agg_ref, ce_ref, b_ref, o_ref):
    a = agg_ref[...]
    ssum = a[0, :, 0:K] + a[1, :, 0:K]
    logits = dinv_ref[...] * (ssum + g_ref[:, 0:K]) + b_ref[...]
    m = jnp.max(logits, axis=1, keepdims=True)
    e = jnp.exp(logits - m)
    sm = e / jnp.sum(e, axis=1, keepdims=True)
    p = jnp.dot(sm, ce_ref[...], preferred_element_type=jnp.float32)
    o_ref[...] = x_ref[...] + p


def _tc_final(x, g, dinv, agg, ce, b2):
    return pl.pallas_call(
        _tc_final_body,
        grid=(G_TC,),
        in_specs=[
            pl.BlockSpec((BLK, D), lambda i: (i, 0)),
            pl.BlockSpec((BLK, 128), lambda i: (i, 0)),
            pl.BlockSpec((BLK, 1), lambda i: (i, 0)),
            pl.BlockSpec((NC, BLK, 128), lambda i: (0, i, 0)),
            pl.BlockSpec((K, D), lambda i: (0, 0)),
            pl.BlockSpec((1, K), lambda i: (0, 0)),
        ],
        out_specs=pl.BlockSpec((BLK, D), lambda i: (i, 0)),
        out_shape=jax.ShapeDtypeStruct((N, D), jnp.float32),
    )(x, g, dinv, agg, ce, b2)


@jax.jit
def kernel(x, edge_index, batch, cluster_emb, W, b):
    del batch
    e3 = edge_index.reshape(2, NCHUNK, CH)

    deg2 = _sc_deg(e3)
    g, dinv = _tc_prep(x, deg2, W, cluster_emb)
    g8 = g.reshape(N_PAD * 8, K)
    agg = _sc_edges(e3, g8)
    return _tc_final(x, g, dinv, agg, cluster_emb, b.reshape(1, K))


# speedup vs baseline: 89.4555x; 1.0263x over previous
import functools

import jax
import jax.numpy as jnp
from jax import lax
from jax.experimental import pallas as pl
from jax.experimental.pallas import tpu as pltpu
from jax.experimental.pallas import tpu_sc as plsc

N = 10000
E = 320000
D = 128
K = 16

NC = 2
NS = 16
NW = NC * NS

CH = 128
NCHUNK = E // CH
BASE_STEPS = NCHUNK // NW
EXTRA = NCHUNK - BASE_STEPS * NW
MAX_STEPS = BASE_STEPS + 1

N_PAD = 10240
RPT = N_PAD // NS
BLK = 1280
G_TC = N_PAD // BLK

_mesh = plsc.VectorSubcoreMesh(core_axis_name="c", subcore_axis_name="s")
_sc_params = pltpu.CompilerParams(use_tc_tiling_on_sc=False)


def _chunk_start(wid):
    return BASE_STEPS * wid + jnp.minimum(wid, EXTRA)


@functools.partial(
    pl.kernel,
    out_type=jax.ShapeDtypeStruct((NC, N_PAD), jnp.float32),
    mesh=_mesh,
    compiler_params=_sc_params,
    scratch_types=[
        pltpu.VMEM((MAX_STEPS, CH), jnp.int32),
        pltpu.VMEM((CH,), jnp.float32),
        pltpu.VMEM((RPT,), jnp.float32),
        pltpu.VMEM_SHARED((N_PAD,), jnp.float32),
        pltpu.SemaphoreType.DMA,
    ],
)
def _sc_deg(e3_hbm, out_hbm, idx_v, ones_v, zb_v, deg_sh, sem):
    c = lax.axis_index("c")
    s = lax.axis_index("s")
    wid = c * NS + s
    start = _chunk_start(wid)
    nsteps = BASE_STEPS + jnp.where(wid < EXTRA, 1, 0)
    pltpu.sync_copy(e3_hbm.at[1, pl.ds(start, BASE_STEPS)],
                    idx_v.at[pl.ds(0, BASE_STEPS)])

    @pl.when(wid < EXTRA)
    def _():
        pltpu.sync_copy(e3_hbm.at[1, pl.ds(start + BASE_STEPS, 1)],
                        idx_v.at[pl.ds(BASE_STEPS, 1)])

    for i in range(CH // 16):
        ones_v[pl.ds(i * 16, 16)] = jnp.ones((16,), jnp.float32)
    for i in range(RPT // 16):
        zb_v[pl.ds(i * 16, 16)] = jnp.zeros((16,), jnp.float32)
    pltpu.sync_copy(zb_v, deg_sh.at[pl.ds(s * RPT, RPT)])
    plsc.subcore_barrier()

    GK = 13
    GROUPS = BASE_STEPS // GK
    for gidx in range(GROUPS):
        for b in range(GK):
            pltpu.async_copy(ones_v, deg_sh.at[idx_v.at[gidx * GK + b]],
                             sem, add=True)
        if gidx >= 2:
            for b in range(GK):
                pltpu.make_async_copy(
                    ones_v, deg_sh.at[idx_v.at[b]], sem).wait()
    for b in range(2 * GK):
        pltpu.make_async_copy(ones_v, deg_sh.at[idx_v.at[0]], sem).wait()

    @pl.when(wid < EXTRA)
    def _():
        pltpu.sync_copy(ones_v, deg_sh.at[idx_v.at[BASE_STEPS]], add=True)

    plsc.subcore_barrier()
    pltpu.sync_copy(deg_sh.at[pl.ds(s * RPT, RPT)],
                    out_hbm.at[c, pl.ds(s * RPT, RPT)])


@functools.partial(
    pl.kernel,
    out_type=jax.ShapeDtypeStruct((N_PAD, 128), jnp.float32),
    mesh=_mesh,
    compiler_params=_sc_params,
    scratch_types=[
        pltpu.VMEM((MAX_STEPS, CH), jnp.int32),
        pltpu.VMEM((MAX_STEPS, CH), jnp.int32),
        pltpu.VMEM((3, 13, CH, K), jnp.float32),
        pltpu.VMEM_SHARED((N_PAD, K), jnp.float32),
        pltpu.SemaphoreType.DMA,
        pltpu.SemaphoreType.DMA,
        pltpu.SemaphoreType.DMA,
        pltpu.SemaphoreType.DMA,
    ],
)
def _sc_edges(e3_hbm, g8_hbm, out_hbm, srcv, dstv, rows_v, agg_sh,
              gsem, ssem0, ssem1, ssem2):
    c = lax.axis_index("c")
    s = lax.axis_index("s")
    wid = c * NS + s
    start = _chunk_start(wid)
    nsteps = BASE_STEPS + jnp.where(wid < EXTRA, 1, 0)
    pltpu.sync_copy(e3_hbm.at[0, pl.ds(start, BASE_STEPS)],
                    srcv.at[pl.ds(0, BASE_STEPS)])
    pltpu.sync_copy(e3_hbm.at[1, pl.ds(start, BASE_STEPS)],
                    dstv.at[pl.ds(0, BASE_STEPS)])

    @pl.when(wid < EXTRA)
    def _():
        pltpu.sync_copy(e3_hbm.at[0, pl.ds(start + BASE_STEPS, 1)],
                        srcv.at[pl.ds(BASE_STEPS, 1)])
        pltpu.sync_copy(e3_hbm.at[1, pl.ds(start + BASE_STEPS, 1)],
                        dstv.at[pl.ds(BASE_STEPS, 1)])

    def scale(j, carry):
        for i in range(CH // 16):
            sl = pl.ds(i * 16, 16)
            srcv[j, sl] = lax.shift_left(srcv[j, sl], 3)
        return carry

    lax.fori_loop(0, MAX_STEPS, scale, 0)

    for i in range(CH):
        rows_v[0, 0, i] = jnp.zeros((K,), jnp.float32)
    for t in range(RPT // CH):
        pltpu.sync_copy(rows_v.at[0, 0],
                        agg_sh.at[pl.ds(s * RPT + t * CH, CH)])
    plsc.subcore_barrier()

    GK = 13
    ssems = (ssem0, ssem1, ssem2)

    def super_body(t, carry):
        for p in range(3):
            gbase = (3 * t + p) * GK

            @pl.when(t > 0)
            def _():
                for b in range(GK):
                    pltpu.make_async_copy(
                        rows_v.at[p, b], agg_sh.at[dstv.at[b]],
                        ssems[p]).wait()

            for b in range(GK):
                pltpu.async_copy(g8_hbm.at[srcv.at[gbase + b]],
                                 rows_v.at[p, b], gsem)
            for b in range(GK):
                pltpu.make_async_copy(g8_hbm.at[srcv.at[gbase + b]],
                                      rows_v.at[p, b], gsem).wait()
            for b in range(GK):
                pltpu.async_copy(rows_v.at[p, b],
                                 agg_sh.at[dstv.at[gbase + b]],
                                 ssems[p], add=True)
        return carry

    lax.fori_loop(0, BASE_STEPS // (3 * GK), super_body, 0)
    for p in range(3):
        for b in range(GK):
            pltpu.make_async_copy(rows_v.at[p, b], agg_sh.at[dstv.at[b]],
                                  ssems[p]).wait()

    @pl.when(wid < EXTRA)
    def _():
        pltpu.sync_copy(g8_hbm.at[srcv.at[BASE_STEPS]], rows_v.at[0, 0])
        pltpu.sync_copy(rows_v.at[0, 0], agg_sh.at[dstv.at[BASE_STEPS]],
                        add=True)

    plsc.subcore_barrier()
    sl = pl.ds(s * RPT, RPT)
    pltpu.sync_copy(agg_sh.at[sl], out_hbm.at[sl, pl.ds(c * K, K)])


def _tc_prep_body(x_ref, dp_ref, w_ref, ce_ref, g_ref):
    csum = jnp.sum(ce_ref[...], axis=0, keepdims=True)
    xi = x_ref[...] + csum
    h = jnp.dot(xi, w_ref[...], preferred_element_type=jnp.float32)
    deg = 1.0 + lax.dot_general(
        dp_ref[...], jnp.ones((NC, 1), jnp.float32),
        (((0,), (0,)), ((), ())), preferred_element_type=jnp.float32)
    dinv = lax.rsqrt(deg)
    g_ref[...] = jnp.concatenate(
        [h * dinv, dinv, jnp.zeros((BLK, 128 - K - 1), jnp.float32)], axis=1)


def _tc_prep(x, dp, w, ce):
    return pl.pallas_call(
        _tc_prep_body,
        grid=(G_TC,),
        in_specs=[
            pl.BlockSpec((BLK, D), lambda i: (i, 0)),
            pl.BlockSpec((NC, BLK), lambda i: (0, i)),
            pl.BlockSpec((D, K), lambda i: (0, 0)),
            pl.BlockSpec((K, D), lambda i: (0, 0)),
        ],
        out_specs=pl.BlockSpec((BLK, 128), lambda i: (i, 0)),
        out_shape=jax.ShapeDtypeStruct((N_PAD, 128), jnp.float32),
    )(x, dp, w, ce)


def _tc_final_body(x_ref, g_ref, agg_ref, ce_ref, b_ref, o_ref):
    a = agg_ref[...]
    ssum = a[:, 0:K] + a[:, K:2 * K]
    dinv = g_ref[:, K:K + 1]
    logits = dinv * (ssum + g_ref[:, 0:K]) + b_ref[...]
    m = jnp.max(logits, axis=1, keepdims=True)
    e = jnp.exp(logits - m)
    sm = e / jnp.sum(e, axis=1, keepdims=True)
    p = jnp.dot(sm, ce_ref[...], preferred_element_type=jnp.float32)
    o_ref[...] = x_ref[...] + p


def _tc_final(x, g, agg, ce, b2):
    return pl.pallas_call(
        _tc_final_body,
        grid=(G_TC,),
        in_specs=[
            pl.BlockSpec((BLK, D), lambda i: (i, 0)),
            pl.BlockSpec((BLK, 128), lambda i: (i, 0)),
            pl.BlockSpec((BLK, 128), lambda i: (i, 0)),
            pl.BlockSpec((K, D), lambda i: (0, 0)),
            pl.BlockSpec((1, K), lambda i: (0, 0)),
        ],
        out_specs=pl.BlockSpec((BLK, D), lambda i: (i, 0)),
        out_shape=jax.ShapeDtypeStruct((N, D), jnp.float32),
    )(x, g, agg, ce, b2)


@jax.jit
def kernel(x, edge_index, batch, cluster_emb, W, b):
    del batch
    e3 = edge_index.reshape(2, NCHUNK, CH)

    deg2 = _sc_deg(e3)
    g = _tc_prep(x, deg2, W, cluster_emb)
    g8 = g.reshape(N_PAD * 8, K)
    agg = _sc_edges(e3, g8)
    return _tc_final(x, g, agg, cluster_emb, b.reshape(1, K))
